# baseline (speedup 1.0000x reference)
import jax
import jax.numpy as jnp
import numpy as np
from jax.experimental import pallas as pl
from jax.experimental.pallas import tpu as pltpu

D_MODEL = 4096
N_HEADS = 32
N_KV_HEADS = 8
HEAD_DIM = 128
GROUP = N_HEADS // N_KV_HEADS
B = 32
T = 4
NTOK = B * T

GCOLS = GROUP * HEAD_DIM
NCHUNK = 4
OCHUNK = D_MODEL // NCHUNK


def _attn_kernel(x_ref, wq_ref, wk_ref, wv_ref, wo_ref, out_ref, o_ref):
    g = pl.program_id(0)
    c = pl.program_id(1)

    @pl.when(c == 0)
    def _attend():
        x = x_ref[...]
        k = jnp.dot(x, wk_ref[...], preferred_element_type=jnp.float32)
        v = jnp.dot(x, wv_ref[...], preferred_element_type=jnp.float32)

        row = jax.lax.broadcasted_iota(jnp.int32, (NTOK, NTOK), 0)
        col = jax.lax.broadcasted_iota(jnp.int32, (NTOK, NTOK), 1)
        valid = (col <= row) & (col >= (row // T) * T)

        scale = jnp.float32(1.0 / np.sqrt(HEAD_DIM))
        for h in range(GROUP):
            qh = jnp.dot(
                x,
                wq_ref[:, h * HEAD_DIM:(h + 1) * HEAD_DIM],
                preferred_element_type=jnp.float32,
            )
            s = jax.lax.dot_general(
                qh, k, (((1,), (1,)), ((), ())),
                preferred_element_type=jnp.float32,
            ) * scale
            s = jnp.where(valid, s, jnp.float32(-1e30))
            m = jnp.max(s, axis=1, keepdims=True)
            p = jnp.exp(s - m)
            p = p / jnp.sum(p, axis=1, keepdims=True)
            o_ref[:, h * HEAD_DIM:(h + 1) * HEAD_DIM] = jnp.dot(
                p, v, preferred_element_type=jnp.float32)

    contrib = jnp.dot(o_ref[...], wo_ref[...],
                      preferred_element_type=jnp.float32)
    ocol = pl.multiple_of(c * OCHUNK, OCHUNK)

    @pl.when(g == 0)
    def _init():
        out_ref[:, pl.ds(ocol, OCHUNK)] = contrib

    @pl.when(g > 0)
    def _accum():
        out_ref[:, pl.ds(ocol, OCHUNK)] += contrib


@jax.jit
def kernel(x, Wq, Wk, Wv, Wo):
    Bx, Tx, Dx = x.shape
    xf = x.reshape(Bx * Tx, Dx)
    out = pl.pallas_call(
        _attn_kernel,
        grid=(N_KV_HEADS, NCHUNK),
        in_specs=[
            pl.BlockSpec((NTOK, D_MODEL), lambda g, c: (0, 0)),
            pl.BlockSpec((D_MODEL, GCOLS), lambda g, c: (0, g)),
            pl.BlockSpec((D_MODEL, HEAD_DIM), lambda g, c: (0, g)),
            pl.BlockSpec((D_MODEL, HEAD_DIM), lambda g, c: (0, g)),
            pl.BlockSpec((GCOLS, OCHUNK), lambda g, c: (g, c)),
        ],
        out_specs=pl.BlockSpec((NTOK, D_MODEL), lambda g, c: (0, 0)),
        out_shape=jax.ShapeDtypeStruct((NTOK, D_MODEL), jnp.float32),
        scratch_shapes=[
            pltpu.VMEM((NTOK, GCOLS), jnp.float32),
        ],
    )(xf, Wq, Wk, Wv, Wo)
    return out.reshape(Bx, Tx, Dx)


# speedup vs baseline: 1.5710x; 1.5710x over previous
import jax
import jax.numpy as jnp
import numpy as np
from jax.experimental import pallas as pl
from jax.experimental.pallas import tpu as pltpu

D_MODEL = 4096
N_HEADS = 32
N_KV_HEADS = 8
HEAD_DIM = 128
GROUP = N_HEADS // N_KV_HEADS
B = 32
T = 4
NTOK = B * T

GCOLS = GROUP * HEAD_DIM


def _attn_group_kernel(x_ref, wq_ref, wk_ref, wv_ref, wo_hbm, out_ref,
                       wo_buf, sem):
    g = pl.program_id(0)
    slot = jax.lax.rem(g, 2)
    nslot = jax.lax.rem(g + 1, 2)

    @pl.when(g == 0)
    def _first_wo():
        pltpu.make_async_copy(
            wo_hbm.at[pl.ds(0, GCOLS), :], wo_buf.at[0], sem.at[0],
        ).start()

    @pl.when(g < N_KV_HEADS - 1)
    def _next_wo():
        pltpu.make_async_copy(
            wo_hbm.at[pl.ds((g + 1) * GCOLS, GCOLS), :],
            wo_buf.at[nslot], sem.at[nslot],
        ).start()

    x = x_ref[...]
    k = jnp.dot(x, wk_ref[...], preferred_element_type=jnp.float32)
    v = jnp.dot(x, wv_ref[...], preferred_element_type=jnp.float32)

    row = jax.lax.broadcasted_iota(jnp.int32, (NTOK, NTOK), 0)
    col = jax.lax.broadcasted_iota(jnp.int32, (NTOK, NTOK), 1)
    valid = (col <= row) & (col >= (row // T) * T)

    scale = jnp.float32(1.0 / np.sqrt(HEAD_DIM))
    os = []
    for h in range(GROUP):
        qh = jnp.dot(
            x,
            wq_ref[:, h * HEAD_DIM:(h + 1) * HEAD_DIM],
            preferred_element_type=jnp.float32,
        )
        s = jax.lax.dot_general(
            qh, k, (((1,), (1,)), ((), ())),
            preferred_element_type=jnp.float32,
        ) * scale
        s = jnp.where(valid, s, jnp.float32(-1e30))
        m = jnp.max(s, axis=1, keepdims=True)
        p = jnp.exp(s - m)
        p = p / jnp.sum(p, axis=1, keepdims=True)
        os.append(jnp.dot(p, v, preferred_element_type=jnp.float32))

    pltpu.make_async_copy(
        wo_hbm.at[pl.ds(g * GCOLS, GCOLS), :],
        wo_buf.at[slot], sem.at[slot],
    ).wait()

    wo = wo_buf.at[slot]
    acc = jnp.dot(os[0], wo[0 * HEAD_DIM:1 * HEAD_DIM, :],
                  preferred_element_type=jnp.float32)
    for h in range(1, GROUP):
        acc += jnp.dot(os[h], wo[h * HEAD_DIM:(h + 1) * HEAD_DIM, :],
                       preferred_element_type=jnp.float32)

    @pl.when(g == 0)
    def _init():
        out_ref[...] = acc

    @pl.when(g > 0)
    def _accum():
        out_ref[...] += acc


@jax.jit
def kernel(x, Wq, Wk, Wv, Wo):
    Bx, Tx, Dx = x.shape
    xf = x.reshape(Bx * Tx, Dx)
    out = pl.pallas_call(
        _attn_group_kernel,
        grid=(N_KV_HEADS,),
        in_specs=[
            pl.BlockSpec((NTOK, D_MODEL), lambda g: (0, 0)),
            pl.BlockSpec((D_MODEL, GCOLS), lambda g: (0, g)),
            pl.BlockSpec((D_MODEL, HEAD_DIM), lambda g: (0, g)),
            pl.BlockSpec((D_MODEL, HEAD_DIM), lambda g: (0, g)),
            pl.BlockSpec(memory_space=pl.ANY),
        ],
        out_specs=pl.BlockSpec((NTOK, D_MODEL), lambda g: (0, 0)),
        out_shape=jax.ShapeDtypeStruct((NTOK, D_MODEL), jnp.float32),
        scratch_shapes=[
            pltpu.VMEM((2, GCOLS, D_MODEL), jnp.float32),
            pltpu.SemaphoreType.DMA((2,)),
        ],
    )(xf, Wq, Wk, Wv, Wo)
    return out.reshape(Bx, Tx, Dx)
